# Initial kernel scaffold; baseline (speedup 1.0000x reference)
import jax
import jax.numpy as jnp
from jax.experimental import pallas as pl


def kernel(max_offset, q, k, v, H, W, win_r, attn_num, attn_type, scale):
    raise NotImplementedError("write your pallas kernel here")


# speedup vs baseline: 66.5395x; 66.5395x over previous
import functools

import jax
import jax.numpy as jnp
from jax import lax
from jax.experimental import pallas as pl
from jax.experimental.pallas import tpu as pltpu
from jax.experimental.pallas import tpu_sc as plsc

_HS = 64
_WS = 96
_NHEADS = 6
_D = 32
_A = 25
_L = 16

_NC = 2
_NS = 16
_NW = _NC * _NS


def _round_half_even_i32(x):
    xc = jnp.clip(x, -1000.0, 1000.0)
    t = xc.astype(jnp.int32)
    fr = xc - t.astype(jnp.float32)
    af = jnp.abs(fr)
    odd = (t & 1) != 0
    one = jnp.ones_like(t)
    zero = jnp.zeros_like(t)
    adj = jnp.where(af > 0.5, one, jnp.where(af < 0.5, zero,
                                             jnp.where(odd, one, zero)))
    sgn = jnp.where(fr < 0.0, -one, one)
    return t + sgn * adj


def _sc_body(oy_hbm, ox_hbm, q_hbm, k_hbm, v_hbm, scale_hbm,
             out_hbm, attn_hbm,
             idx_v, krows_v, vrows_v, oy_v, ox_v, q_v, scale_v, attn_v, out_v,
             sem_k, sem_v, n_pairs, chunks_per_tile):
    cid = lax.axis_index("c")
    sid = lax.axis_index("s")
    wid = sid * _NC + cid
    pairs_per_tile = n_pairs // _NW

    pltpu.sync_copy(scale_hbm, scale_v)
    scale_vec = scale_v[...]
    lane = lax.iota(jnp.int32, _L)
    zeros_i = jnp.zeros((_L,), jnp.int32)
    ones_i = jnp.ones((_L,), jnp.int32)
    neg_big = jnp.full((_L,), -1e30, jnp.float32)

    def chunk_body(ci, carry):
        p0 = wid * pairs_per_tile + ci * _L
        pltpu.sync_copy(oy_hbm.at[pl.ds(p0, _L)], oy_v)
        pltpu.sync_copy(ox_hbm.at[pl.ds(p0, _L)], ox_v)
        pltpu.sync_copy(q_hbm.at[pl.ds(p0, _L), :], q_v)

        ci16 = ci * _L
        qq = (ci16 * 10923) >> 16
        rr = ci16 - _NHEADS * qq
        t = rr + lane
        dcarry = (jnp.where(t >= 6, ones_i, zeros_i)
                  + jnp.where(t >= 12, ones_i, zeros_i)
                  + jnp.where(t >= 18, ones_i, zeros_i))
        head = t - _NHEADS * dcarry
        ph = wid * (pairs_per_tile // _NHEADS) + qq + dcarry
        b_vec = jnp.where(ph >= _HS * _WS, ones_i, zeros_i)
        n_vec = ph - b_vec * (_HS * _WS)
        gy = ((n_vec >> 5) * 21846) >> 16
        gx = n_vec - gy * _WS

        oy = oy_v[...]
        ox = ox_v[...]
        cy = jnp.clip(gy + _round_half_even_i32(oy), 0, _HS - 1)
        cx = jnp.clip(gx + _round_half_even_i32(ox), 0, _WS - 1)
        base = b_vec * (_HS * _WS * _NHEADS) + head

        for a in range(_A):
            dy = a // 5 - 2
            dx = a % 5 - 2
            ky = jnp.clip(cy + dy, 0, _HS - 1)
            kx = jnp.clip(cx + dx, 0, _WS - 1)
            idx_v[a // 5, pl.ds((a % 5) * _L, _L)] = \
                base + (ky * _WS + kx) * _NHEADS

        copies = []
        for j in range(5):
            cpk = pltpu.make_async_copy(k_hbm.at[idx_v.at[j]],
                                        krows_v.at[j], sem_k)
            cpv = pltpu.make_async_copy(v_hbm.at[idx_v.at[j]],
                                        vrows_v.at[j], sem_v)
            cpk.start()
            cpv.start()
            copies.append((cpk, cpv))
        for cpk, cpv in copies:
            cpk.wait()
            cpv.wait()

        def pair_body(pi, carry2):
            q0 = q_v[pi, pl.ds(0, _L)]
            q1 = q_v[pi, pl.ds(_L, _L)]
            l0 = neg_big
            l1 = neg_big
            for a in range(_A):
                k0 = krows_v[a // 5, (a % 5) * _L + pi, pl.ds(0, _L)]
                k1 = krows_v[a // 5, (a % 5) * _L + pi, pl.ds(_L, _L)]
                s = jnp.sum(jnp.abs(q0 - k0) + jnp.abs(q1 - k1))
                if a < _L:
                    l0 = jnp.where(lane == a, -s, l0)
                else:
                    l1 = jnp.where(lane == (a - _L), -s, l1)
            l0 = l0 * scale_vec
            l1 = jnp.where(lane < 9, l1 * scale_vec, neg_big)
            mm = jnp.max(jnp.maximum(l0, l1))
            e0 = jnp.exp(l0 - mm)
            e1 = jnp.exp(l1 - mm)
            ssum = jnp.sum(e0 + e1)
            a0 = e0 / ssum
            a1 = e1 / ssum
            attn_v[pi, pl.ds(0, _L)] = a0
            plsc.store_scatter(attn_v,
                               [jnp.full((_L,), pi, jnp.int32), _L + lane],
                               a1, mask=lane < 9)

            acc0 = jnp.zeros((_L,), jnp.float32)
            acc1 = jnp.zeros((_L,), jnp.float32)
            for a in range(_A):
                w = a0[a] if a < _L else a1[a - _L]
                acc0 = acc0 + w * vrows_v[a // 5, (a % 5) * _L + pi,
                                          pl.ds(0, _L)]
                acc1 = acc1 + w * vrows_v[a // 5, (a % 5) * _L + pi,
                                          pl.ds(_L, _L)]
            out_v[pi, pl.ds(0, _L)] = acc0
            out_v[pi, pl.ds(_L, _L)] = acc1
            return carry2

        lax.fori_loop(0, _L, pair_body, 0)

        pltpu.sync_copy(out_v, out_hbm.at[pl.ds(p0, _L), :])
        pltpu.sync_copy(attn_v, attn_hbm.at[pl.ds(p0, _L), :])
        return carry

    lax.fori_loop(0, chunks_per_tile, chunk_body, 0)


@functools.partial(jax.jit, static_argnames=())
def _mf_forward_sc(oy_flat, ox_flat, q_flat, k_flat, v_flat, scale_vec):
    n_pairs = q_flat.shape[0]
    chunks_per_tile = n_pairs // (_NW * _L)
    mesh = plsc.VectorSubcoreMesh(core_axis_name="c", subcore_axis_name="s")
    body = functools.partial(_sc_body, n_pairs=n_pairs,
                             chunks_per_tile=chunks_per_tile)
    f = pl.kernel(
        body,
        out_type=(
            jax.ShapeDtypeStruct((n_pairs, _D), jnp.float32),
            jax.ShapeDtypeStruct((n_pairs, _A), jnp.float32),
        ),
        mesh=mesh,
        compiler_params=pltpu.CompilerParams(
            needs_layout_passes=False,
            use_tc_tiling_on_sc=False,
        ),
        scratch_types=[
            pltpu.VMEM((5, 5 * _L), jnp.int32),
            pltpu.VMEM((5, 5 * _L, _D), jnp.float32),
            pltpu.VMEM((5, 5 * _L, _D), jnp.float32),
            pltpu.VMEM((_L,), jnp.float32),
            pltpu.VMEM((_L,), jnp.float32),
            pltpu.VMEM((_L, _D), jnp.float32),
            pltpu.VMEM((_L,), jnp.float32),
            pltpu.VMEM((_L, _A), jnp.float32),
            pltpu.VMEM((_L, _D), jnp.float32),
            pltpu.SemaphoreType.DMA,
            pltpu.SemaphoreType.DMA,
        ],
    )
    return f(oy_flat, ox_flat, q_flat, k_flat, v_flat, scale_vec)


def kernel(max_offset, q, k, v, H, W, win_r, attn_num, attn_type=1.0,
           scale=1.0):
    B, N, C = q.shape
    h = max_offset.shape[2]
    n_pairs = B * N * h
    mo_flat = max_offset.reshape(n_pairs, 2)
    oy_flat = mo_flat[:, 0]
    ox_flat = mo_flat[:, 1]
    q_flat = q.reshape(n_pairs, _D)
    k_flat = k.reshape(n_pairs, _D)
    v_flat = v.reshape(n_pairs, _D)
    scale_vec = jnp.full((_L,), 1.0, jnp.float32) * jnp.asarray(
        scale, jnp.float32)
    out_flat, attn_flat = _mf_forward_sc(oy_flat, ox_flat, q_flat, k_flat,
                                         v_flat, scale_vec)
    output = out_flat.reshape(B, N, C)
    attn_out = attn_flat.reshape(B, N, h, _A)
    return output, attn_out


# speedup vs baseline: 95.9717x; 1.4423x over previous
import functools

import jax
import jax.numpy as jnp
from jax import lax
from jax.experimental import pallas as pl
from jax.experimental.pallas import tpu as pltpu
from jax.experimental.pallas import tpu_sc as plsc

_HS = 64
_WS = 96
_NHEADS = 6
_D = 32
_A = 25
_L = 16

_NC = 2
_NS = 16
_NW = _NC * _NS


def _round_half_even_i32(x):
    xc = jnp.clip(x, -1000.0, 1000.0)
    t = xc.astype(jnp.int32)
    fr = xc - t.astype(jnp.float32)
    af = jnp.abs(fr)
    odd = (t & 1) != 0
    one = jnp.ones_like(t)
    zero = jnp.zeros_like(t)
    adj = jnp.where(af > 0.5, one, jnp.where(af < 0.5, zero,
                                             jnp.where(odd, one, zero)))
    sgn = jnp.where(fr < 0.0, -one, one)
    return t + sgn * adj


def _sc_body(mo_hbm, q_hbm, k_hbm, v_hbm, scale_hbm,
             out_hbm, attn_hbm,
             idx_v, krows_v, vrows_v, mo_v, q_v, scale_v, attn_v, out_v,
             sem_k0, sem_k1, sem_v0, sem_v1, sem_q0, sem_q1,
             sem_o0, sem_o1, sem_a0, sem_a1,
             n_pairs, chunks_per_tile):
    cid = lax.axis_index("c")
    sid = lax.axis_index("s")
    wid = sid * _NC + cid
    pairs_per_tile = n_pairs // _NW

    sem_k = (sem_k0, sem_k1)
    sem_v = (sem_v0, sem_v1)
    sem_q = (sem_q0, sem_q1)
    sem_o = (sem_o0, sem_o1)
    sem_a = (sem_a0, sem_a1)

    pltpu.sync_copy(scale_hbm, scale_v)
    scale_vec = scale_v[...]
    lane = lax.iota(jnp.int32, _L)
    zeros_i = jnp.zeros((_L,), jnp.int32)
    ones_i = jnp.ones((_L,), jnp.int32)
    neg_big = jnp.full((_L,), -1e30, jnp.float32)

    def gather_copies(ci, slot):
        p0 = wid * pairs_per_tile + ci * _L
        cps = []
        for j in range(5):
            cps.append(pltpu.make_async_copy(
                k_hbm.at[idx_v.at[slot, j]], krows_v.at[slot, j],
                sem_k[slot]))
            cps.append(pltpu.make_async_copy(
                v_hbm.at[idx_v.at[slot, j]], vrows_v.at[slot, j],
                sem_v[slot]))
        cps.append(pltpu.make_async_copy(
            q_hbm.at[pl.ds(p0, _L), :], q_v.at[slot], sem_q[slot]))
        return cps

    def out_copies(ci, slot):
        p0 = wid * pairs_per_tile + ci * _L
        return [
            pltpu.make_async_copy(out_v.at[slot],
                                  out_hbm.at[pl.ds(p0, _L), :], sem_o[slot]),
            pltpu.make_async_copy(attn_v.at[slot],
                                  attn_hbm.at[pl.ds(p0, _L), :], sem_a[slot]),
        ]

    def fire_chunk(ci, slot):
        p0 = wid * pairs_per_tile + ci * _L
        pltpu.sync_copy(mo_hbm.at[pl.ds(p0, _L), :], mo_v)

        ci16 = ci * _L
        qq = (ci16 * 10923) >> 16
        rr = ci16 - _NHEADS * qq
        t = rr + lane
        dcarry = (jnp.where(t >= 6, ones_i, zeros_i)
                  + jnp.where(t >= 12, ones_i, zeros_i)
                  + jnp.where(t >= 18, ones_i, zeros_i))
        head = t - _NHEADS * dcarry
        ph = wid * (pairs_per_tile // _NHEADS) + qq + dcarry
        b_vec = jnp.where(ph >= _HS * _WS, ones_i, zeros_i)
        n_vec = ph - b_vec * (_HS * _WS)
        gy = ((n_vec >> 5) * 21846) >> 16
        gx = n_vec - gy * _WS

        oy = plsc.load_gather(mo_v, [lane, zeros_i])
        ox = plsc.load_gather(mo_v, [lane, ones_i])
        cy = jnp.clip(gy + _round_half_even_i32(oy), 0, _HS - 1)
        cx = jnp.clip(gx + _round_half_even_i32(ox), 0, _WS - 1)
        base = b_vec * (_HS * _WS * _NHEADS) + head

        for a in range(_A):
            dy = a // 5 - 2
            dx = a % 5 - 2
            ky = jnp.clip(cy + dy, 0, _HS - 1)
            kx = jnp.clip(cx + dx, 0, _WS - 1)
            idx_v[slot, a // 5, pl.ds((a % 5) * _L, _L)] = \
                base + (ky * _WS + kx) * _NHEADS

        for cp in gather_copies(ci, slot):
            cp.start()

    def compute_chunk(ci, slot):
        def pair_body(pi, carry2):
            q0 = q_v[slot, pi, pl.ds(0, _L)]
            q1 = q_v[slot, pi, pl.ds(_L, _L)]
            l0 = neg_big
            l1 = neg_big
            for a in range(_A):
                k0 = krows_v[slot, a // 5, (a % 5) * _L + pi, pl.ds(0, _L)]
                k1 = krows_v[slot, a // 5, (a % 5) * _L + pi, pl.ds(_L, _L)]
                s = jnp.sum(jnp.abs(q0 - k0) + jnp.abs(q1 - k1))
                if a < _L:
                    l0 = jnp.where(lane == a, -s, l0)
                else:
                    l1 = jnp.where(lane == (a - _L), -s, l1)
            l0 = l0 * scale_vec
            l1 = jnp.where(lane < 9, l1 * scale_vec, neg_big)
            mm = jnp.max(jnp.maximum(l0, l1))
            e0 = jnp.exp(l0 - mm)
            e1 = jnp.exp(l1 - mm)
            ssum = jnp.sum(e0 + e1)
            a0 = e0 / ssum
            a1 = e1 / ssum
            attn_v[slot, pi, pl.ds(0, _L)] = a0
            plsc.store_scatter(attn_v.at[slot],
                               [jnp.full((_L,), pi, jnp.int32), _L + lane],
                               a1, mask=lane < 9)

            acc0 = jnp.zeros((_L,), jnp.float32)
            acc1 = jnp.zeros((_L,), jnp.float32)
            for a in range(_A):
                w = a0[a] if a < _L else a1[a - _L]
                acc0 = acc0 + w * vrows_v[slot, a // 5, (a % 5) * _L + pi,
                                          pl.ds(0, _L)]
                acc1 = acc1 + w * vrows_v[slot, a // 5, (a % 5) * _L + pi,
                                          pl.ds(_L, _L)]
            out_v[slot, pi, pl.ds(0, _L)] = acc0
            out_v[slot, pi, pl.ds(_L, _L)] = acc1
            return carry2

        lax.fori_loop(0, _L, pair_body, 0)
        for cp in out_copies(ci, slot):
            cp.start()

    def wait_gathers(ci, slot):
        for cp in gather_copies(ci, slot):
            cp.wait()

    def wait_outputs(ci, slot):
        for cp in out_copies(ci, slot):
            cp.wait()

    n_iters = chunks_per_tile // 2

    fire_chunk(0, 0)

    def loop_body(i, carry):
        cA = 2 * i
        cB = cA + 1
        fire_chunk(cB, 1)
        wait_gathers(cA, 0)

        @pl.when(i >= 1)
        def _():
            wait_outputs(cA, 0)

        compute_chunk(cA, 0)

        @pl.when(i < n_iters - 1)
        def _():
            fire_chunk(cB + 1, 0)

        wait_gathers(cB, 1)

        @pl.when(i >= 1)
        def _():
            wait_outputs(cB, 1)

        compute_chunk(cB, 1)
        return carry

    lax.fori_loop(0, n_iters, loop_body, 0)
    wait_outputs(chunks_per_tile - 2, 0)
    wait_outputs(chunks_per_tile - 1, 1)


@functools.partial(jax.jit, static_argnames=())
def _mf_forward_sc(mo_flat, q_flat, k_flat, v_flat, scale_vec):
    n_pairs = q_flat.shape[0]
    chunks_per_tile = n_pairs // (_NW * _L)
    assert chunks_per_tile % 2 == 0
    mesh = plsc.VectorSubcoreMesh(core_axis_name="c", subcore_axis_name="s")
    body = functools.partial(_sc_body, n_pairs=n_pairs,
                             chunks_per_tile=chunks_per_tile)
    f = pl.kernel(
        body,
        out_type=(
            jax.ShapeDtypeStruct((n_pairs, _D), jnp.float32),
            jax.ShapeDtypeStruct((n_pairs, _A), jnp.float32),
        ),
        mesh=mesh,
        compiler_params=pltpu.CompilerParams(
            needs_layout_passes=False,
            use_tc_tiling_on_sc=False,
        ),
        scratch_types=[
            pltpu.VMEM((2, 5, 5 * _L), jnp.int32),
            pltpu.VMEM((2, 5, 5 * _L, _D), jnp.float32),
            pltpu.VMEM((2, 5, 5 * _L, _D), jnp.float32),
            pltpu.VMEM((_L, 2), jnp.float32),
            pltpu.VMEM((2, _L, _D), jnp.float32),
            pltpu.VMEM((_L,), jnp.float32),
            pltpu.VMEM((2, _L, _A), jnp.float32),
            pltpu.VMEM((2, _L, _D), jnp.float32),
            pltpu.SemaphoreType.DMA,
            pltpu.SemaphoreType.DMA,
            pltpu.SemaphoreType.DMA,
            pltpu.SemaphoreType.DMA,
            pltpu.SemaphoreType.DMA,
            pltpu.SemaphoreType.DMA,
            pltpu.SemaphoreType.DMA,
            pltpu.SemaphoreType.DMA,
            pltpu.SemaphoreType.DMA,
            pltpu.SemaphoreType.DMA,
        ],
    )
    return f(mo_flat, q_flat, k_flat, v_flat, scale_vec)


def kernel(max_offset, q, k, v, H, W, win_r, attn_num, attn_type=1.0,
           scale=1.0):
    B, N, C = q.shape
    h = max_offset.shape[2]
    n_pairs = B * N * h
    mo_flat = max_offset.reshape(n_pairs, 2)
    q_flat = q.reshape(n_pairs, _D)
    k_flat = k.reshape(n_pairs, _D)
    v_flat = v.reshape(n_pairs, _D)
    scale_vec = jnp.full((_L,), 1.0, jnp.float32) * jnp.asarray(
        scale, jnp.float32)
    out_flat, attn_flat = _mf_forward_sc(mo_flat, q_flat, k_flat, v_flat,
                                         scale_vec)
    output = out_flat.reshape(B, N, C)
    attn_out = attn_flat.reshape(B, N, h, _A)
    return output, attn_out


# speedup vs baseline: 101.8596x; 1.0614x over previous
import functools

import jax
import jax.numpy as jnp
from jax import lax
from jax.experimental import pallas as pl
from jax.experimental.pallas import tpu as pltpu
from jax.experimental.pallas import tpu_sc as plsc

_HS = 64
_WS = 96
_NHEADS = 6
_D = 32
_A = 25
_L = 16

_NC = 2
_NS = 16
_NW = _NC * _NS


def _round_half_even_i32(x):
    xc = jnp.clip(x, -1000.0, 1000.0)
    t = xc.astype(jnp.int32)
    fr = xc - t.astype(jnp.float32)
    af = jnp.abs(fr)
    odd = (t & 1) != 0
    one = jnp.ones_like(t)
    zero = jnp.zeros_like(t)
    adj = jnp.where(af > 0.5, one, jnp.where(af < 0.5, zero,
                                             jnp.where(odd, one, zero)))
    sgn = jnp.where(fr < 0.0, -one, one)
    return t + sgn * adj


def _sc_body(mo_hbm, q_hbm, k_hbm, v_hbm, scale_hbm,
             out_hbm, attn_hbm,
             idx_v, krows_v, vrows_v, mo_v, q_v, scale_v, attn_v, out_v,
             sem_k0, sem_k1, sem_v0, sem_v1, sem_q0, sem_q1,
             sem_o0, sem_o1, sem_a0, sem_a1,
             n_pairs, chunks_per_tile):
    cid = lax.axis_index("c")
    sid = lax.axis_index("s")
    wid = sid * _NC + cid
    pairs_per_tile = n_pairs // _NW

    sem_k = (sem_k0, sem_k1)
    sem_v = (sem_v0, sem_v1)
    sem_q = (sem_q0, sem_q1)
    sem_o = (sem_o0, sem_o1)
    sem_a = (sem_a0, sem_a1)

    pltpu.sync_copy(scale_hbm, scale_v)
    scale_vec = scale_v[...]
    lane = lax.iota(jnp.int32, _L)
    zeros_i = jnp.zeros((_L,), jnp.int32)
    ones_i = jnp.ones((_L,), jnp.int32)
    neg_big = jnp.full((_L,), -1e30, jnp.float32)

    def gather_copies(ci, slot):
        p0 = wid * pairs_per_tile + ci * _L
        cps = []
        for j in range(5):
            cps.append(pltpu.make_async_copy(
                k_hbm.at[idx_v.at[slot, j]], krows_v.at[slot, j],
                sem_k[slot]))
            cps.append(pltpu.make_async_copy(
                v_hbm.at[idx_v.at[slot, j]], vrows_v.at[slot, j],
                sem_v[slot]))
        cps.append(pltpu.make_async_copy(
            q_hbm.at[pl.ds(p0 * _D, _L * _D)], q_v.at[slot], sem_q[slot]))
        return cps

    def out_copies(ci, slot):
        p0 = wid * pairs_per_tile + ci * _L
        return [
            pltpu.make_async_copy(out_v.at[slot],
                                  out_hbm.at[pl.ds(p0 * _D, _L * _D)],
                                  sem_o[slot]),
            pltpu.make_async_copy(attn_v.at[slot],
                                  attn_hbm.at[pl.ds(p0 * _A, _L * _A)],
                                  sem_a[slot]),
        ]

    def fire_chunk(ci, slot):
        p0 = wid * pairs_per_tile + ci * _L
        pltpu.sync_copy(mo_hbm.at[pl.ds(p0, _L), :], mo_v)

        ci16 = ci * _L
        qq = (ci16 * 10923) >> 16
        rr = ci16 - _NHEADS * qq
        t = rr + lane
        dcarry = (jnp.where(t >= 6, ones_i, zeros_i)
                  + jnp.where(t >= 12, ones_i, zeros_i)
                  + jnp.where(t >= 18, ones_i, zeros_i))
        head = t - _NHEADS * dcarry
        ph = wid * (pairs_per_tile // _NHEADS) + qq + dcarry
        b_vec = jnp.where(ph >= _HS * _WS, ones_i, zeros_i)
        n_vec = ph - b_vec * (_HS * _WS)
        gy = ((n_vec >> 5) * 21846) >> 16
        gx = n_vec - gy * _WS

        oy = plsc.load_gather(mo_v, [lane, zeros_i])
        ox = plsc.load_gather(mo_v, [lane, ones_i])
        cy = jnp.clip(gy + _round_half_even_i32(oy), 0, _HS - 1)
        cx = jnp.clip(gx + _round_half_even_i32(ox), 0, _WS - 1)
        base = b_vec * (_HS * _WS * _NHEADS) + head

        for a in range(_A):
            dy = a // 5 - 2
            dx = a % 5 - 2
            ky = jnp.clip(cy + dy, 0, _HS - 1)
            kx = jnp.clip(cx + dx, 0, _WS - 1)
            idx_v[slot, a // 5, pl.ds((a % 5) * _L, _L)] = \
                base + (ky * _WS + kx) * _NHEADS

        for cp in gather_copies(ci, slot):
            cp.start()

    def compute_chunk(ci, slot):
        def pair_body(pi, carry2):
            q0 = q_v[slot, pl.ds(pi * _D, _L)]
            q1 = q_v[slot, pl.ds(pi * _D + _L, _L)]
            l0 = neg_big
            l1 = neg_big
            for a in range(_A):
                k0 = krows_v[slot, a // 5, (a % 5) * _L + pi, pl.ds(0, _L)]
                k1 = krows_v[slot, a // 5, (a % 5) * _L + pi, pl.ds(_L, _L)]
                s = jnp.sum(jnp.abs(q0 - k0) + jnp.abs(q1 - k1))
                if a < _L:
                    l0 = jnp.where(lane == a, -s, l0)
                else:
                    l1 = jnp.where(lane == (a - _L), -s, l1)
            l0 = l0 * scale_vec
            l1 = jnp.where(lane < 9, l1 * scale_vec, neg_big)
            mm = jnp.max(jnp.maximum(l0, l1))
            e0 = jnp.exp(l0 - mm)
            e1 = jnp.exp(l1 - mm)
            ssum = jnp.sum(e0 + e1)
            a0 = e0 / ssum
            a1 = e1 / ssum
            attn_v[slot, pl.ds(pi * _A, _L)] = a0
            plsc.store_scatter(attn_v.at[slot],
                               [pi * _A + _L + lane],
                               a1, mask=lane < 9)

            acc0 = jnp.zeros((_L,), jnp.float32)
            acc1 = jnp.zeros((_L,), jnp.float32)
            for a in range(_A):
                w = a0[a] if a < _L else a1[a - _L]
                acc0 = acc0 + w * vrows_v[slot, a // 5, (a % 5) * _L + pi,
                                          pl.ds(0, _L)]
                acc1 = acc1 + w * vrows_v[slot, a // 5, (a % 5) * _L + pi,
                                          pl.ds(_L, _L)]
            out_v[slot, pl.ds(pi * _D, _L)] = acc0
            out_v[slot, pl.ds(pi * _D + _L, _L)] = acc1
            return carry2

        lax.fori_loop(0, _L, pair_body, 0)
        for cp in out_copies(ci, slot):
            cp.start()

    def wait_gathers(ci, slot):
        for cp in gather_copies(ci, slot):
            cp.wait()

    def wait_outputs(ci, slot):
        for cp in out_copies(ci, slot):
            cp.wait()

    n_iters = chunks_per_tile // 2

    fire_chunk(0, 0)

    def loop_body(i, carry):
        cA = 2 * i
        cB = cA + 1
        fire_chunk(cB, 1)
        wait_gathers(cA, 0)

        @pl.when(i >= 1)
        def _():
            wait_outputs(cA, 0)

        compute_chunk(cA, 0)

        @pl.when(i < n_iters - 1)
        def _():
            fire_chunk(cB + 1, 0)

        wait_gathers(cB, 1)

        @pl.when(i >= 1)
        def _():
            wait_outputs(cB, 1)

        compute_chunk(cB, 1)
        return carry

    lax.fori_loop(0, n_iters, loop_body, 0)
    wait_outputs(chunks_per_tile - 2, 0)
    wait_outputs(chunks_per_tile - 1, 1)


@functools.partial(jax.jit, static_argnames=())
def _mf_forward_sc(mo_flat, q_flat, k_flat, v_flat, scale_vec):
    n_pairs = k_flat.shape[0]
    chunks_per_tile = n_pairs // (_NW * _L)
    assert chunks_per_tile % 2 == 0
    mesh = plsc.VectorSubcoreMesh(core_axis_name="c", subcore_axis_name="s")
    body = functools.partial(_sc_body, n_pairs=n_pairs,
                             chunks_per_tile=chunks_per_tile)
    f = pl.kernel(
        body,
        out_type=(
            jax.ShapeDtypeStruct((n_pairs * _D,), jnp.float32),
            jax.ShapeDtypeStruct((n_pairs * _A,), jnp.float32),
        ),
        mesh=mesh,
        compiler_params=pltpu.CompilerParams(
            needs_layout_passes=False,
            use_tc_tiling_on_sc=False,
        ),
        scratch_types=[
            pltpu.VMEM((2, 5, 5 * _L), jnp.int32),
            pltpu.VMEM((2, 5, 5 * _L, _D), jnp.float32),
            pltpu.VMEM((2, 5, 5 * _L, _D), jnp.float32),
            pltpu.VMEM((_L, 2), jnp.float32),
            pltpu.VMEM((2, _L * _D), jnp.float32),
            pltpu.VMEM((_L,), jnp.float32),
            pltpu.VMEM((2, _L * _A), jnp.float32),
            pltpu.VMEM((2, _L * _D), jnp.float32),
            pltpu.SemaphoreType.DMA,
            pltpu.SemaphoreType.DMA,
            pltpu.SemaphoreType.DMA,
            pltpu.SemaphoreType.DMA,
            pltpu.SemaphoreType.DMA,
            pltpu.SemaphoreType.DMA,
            pltpu.SemaphoreType.DMA,
            pltpu.SemaphoreType.DMA,
            pltpu.SemaphoreType.DMA,
            pltpu.SemaphoreType.DMA,
        ],
    )
    return f(mo_flat, q_flat, k_flat, v_flat, scale_vec)


def kernel(max_offset, q, k, v, H, W, win_r, attn_num, attn_type=1.0,
           scale=1.0):
    B, N, C = q.shape
    h = max_offset.shape[2]
    n_pairs = B * N * h
    mo_flat = max_offset.reshape(n_pairs, 2)
    q_flat = q.reshape(n_pairs * _D)
    k_flat = k.reshape(n_pairs, _D)
    v_flat = v.reshape(n_pairs, _D)
    scale_vec = jnp.full((_L,), 1.0, jnp.float32) * jnp.asarray(
        scale, jnp.float32)
    out_flat, attn_flat = _mf_forward_sc(mo_flat, q_flat, k_flat, v_flat,
                                         scale_vec)
    output = out_flat.reshape(B, N, C)
    attn_out = attn_flat.reshape(B, N, h, _A)
    return output, attn_out


# speedup vs baseline: 117.5724x; 1.1543x over previous
import functools

import jax
import jax.numpy as jnp
from jax import lax
from jax.experimental import pallas as pl
from jax.experimental.pallas import tpu as pltpu
from jax.experimental.pallas import tpu_sc as plsc

_HS = 64
_WS = 96
_NHEADS = 6
_D = 32
_A = 25
_L = 16

_NC = 2
_NS = 16
_NW = _NC * _NS


def _round_half_even_i32(x):
    xc = jnp.clip(x, -1000.0, 1000.0)
    t = xc.astype(jnp.int32)
    fr = xc - t.astype(jnp.float32)
    af = jnp.abs(fr)
    odd = (t & 1) != 0
    one = jnp.ones_like(t)
    zero = jnp.zeros_like(t)
    adj = jnp.where(af > 0.5, one, jnp.where(af < 0.5, zero,
                                             jnp.where(odd, one, zero)))
    sgn = jnp.where(fr < 0.0, -one, one)
    return t + sgn * adj


def _sc_body(mo_hbm, q_hbm, k_hbm, v_hbm, scale_hbm,
             out_hbm, attn_hbm,
             idx_v, krows_v, vrows_v, mo_v, q_v, scale_v, attn_v, out_v,
             sem_k0, sem_k1, sem_v0, sem_v1, sem_q0, sem_q1,
             sem_o0, sem_o1, sem_a0, sem_a1, sem_m0, sem_m1,
             n_pairs, chunks_per_tile):
    cid = lax.axis_index("c")
    sid = lax.axis_index("s")
    wid = sid * _NC + cid
    pairs_per_tile = n_pairs // _NW

    sem_k = (sem_k0, sem_k1)
    sem_v = (sem_v0, sem_v1)
    sem_q = (sem_q0, sem_q1)
    sem_o = (sem_o0, sem_o1)
    sem_a = (sem_a0, sem_a1)
    sem_m = (sem_m0, sem_m1)

    pltpu.sync_copy(scale_hbm, scale_v)
    scale_vec = scale_v[...]
    lane = lax.iota(jnp.int32, _L)
    zeros_i = jnp.zeros((_L,), jnp.int32)
    ones_i = jnp.ones((_L,), jnp.int32)
    neg_big = jnp.full((_L,), -1e30, jnp.float32)

    def gather_copies(ci, slot):
        p0 = wid * pairs_per_tile + ci * _L
        cps = []
        for j in range(5):
            cps.append(pltpu.make_async_copy(
                k_hbm.at[idx_v.at[slot, j]], krows_v.at[slot, j],
                sem_k[slot]))
            cps.append(pltpu.make_async_copy(
                v_hbm.at[idx_v.at[slot, j]], vrows_v.at[slot, j],
                sem_v[slot]))
        cps.append(pltpu.make_async_copy(
            q_hbm.at[pl.ds(p0 * _D, _L * _D)], q_v.at[slot], sem_q[slot]))
        return cps

    def out_copies(ci, slot):
        p0 = wid * pairs_per_tile + ci * _L
        return [
            pltpu.make_async_copy(out_v.at[slot],
                                  out_hbm.at[pl.ds(p0 * _D, _L * _D)],
                                  sem_o[slot]),
            pltpu.make_async_copy(attn_v.at[slot],
                                  attn_hbm.at[pl.ds(p0 * _A, _L * _A)],
                                  sem_a[slot]),
        ]

    def mo_copy(ci, slot):
        p0 = wid * pairs_per_tile + ci * _L
        return pltpu.make_async_copy(mo_hbm.at[pl.ds(p0, _L), :],
                                     mo_v.at[slot], sem_m[slot])

    def fire_chunk(ci, slot):
        mo_copy(ci, slot).wait()
        ci16 = ci * _L
        qq = (ci16 * 10923) >> 16
        rr = ci16 - _NHEADS * qq
        t = rr + lane
        dcarry = (jnp.where(t >= 6, ones_i, zeros_i)
                  + jnp.where(t >= 12, ones_i, zeros_i)
                  + jnp.where(t >= 18, ones_i, zeros_i))
        head = t - _NHEADS * dcarry
        ph = wid * (pairs_per_tile // _NHEADS) + qq + dcarry
        b_vec = jnp.where(ph >= _HS * _WS, ones_i, zeros_i)
        n_vec = ph - b_vec * (_HS * _WS)
        gy = ((n_vec >> 5) * 21846) >> 16
        gx = n_vec - gy * _WS

        oy = plsc.load_gather(mo_v.at[slot], [lane, zeros_i])
        ox = plsc.load_gather(mo_v.at[slot], [lane, ones_i])
        cy = jnp.clip(gy + _round_half_even_i32(oy), 0, _HS - 1)
        cx = jnp.clip(gx + _round_half_even_i32(ox), 0, _WS - 1)
        base = b_vec * (_HS * _WS * _NHEADS) + head

        for a in range(_A):
            dy = a // 5 - 2
            dx = a % 5 - 2
            ky = jnp.clip(cy + dy, 0, _HS - 1)
            kx = jnp.clip(cx + dx, 0, _WS - 1)
            idx_v[slot, a // 5, pl.ds((a % 5) * _L, _L)] = \
                base + (ky * _WS + kx) * _NHEADS

        for cp in gather_copies(ci, slot):
            cp.start()

        @pl.when(ci + 2 < chunks_per_tile)
        def _():
            mo_copy(ci + 2, slot).start()

    def compute_chunk(ci, slot):
        def pair_body(pi, carry2):
            q0 = q_v[slot, pl.ds(pi * _D, _L)]
            q1 = q_v[slot, pl.ds(pi * _D + _L, _L)]
            l0 = neg_big
            l1 = neg_big
            for a in range(_A):
                k0 = krows_v[slot, a // 5, (a % 5) * _L + pi, pl.ds(0, _L)]
                k1 = krows_v[slot, a // 5, (a % 5) * _L + pi, pl.ds(_L, _L)]
                s = jnp.sum(jnp.abs(q0 - k0) + jnp.abs(q1 - k1))
                if a < _L:
                    l0 = jnp.where(lane == a, -s, l0)
                else:
                    l1 = jnp.where(lane == (a - _L), -s, l1)
            l0 = l0 * scale_vec
            l1 = jnp.where(lane < 9, l1 * scale_vec, neg_big)
            mm = jnp.max(jnp.maximum(l0, l1))
            e0 = jnp.exp(l0 - mm)
            e1 = jnp.exp(l1 - mm)
            ssum = jnp.sum(e0 + e1)
            a0 = e0 / ssum
            a1 = e1 / ssum
            attn_v[slot, pl.ds(pi * _A, _L)] = a0
            plsc.store_scatter(attn_v.at[slot],
                               [pi * _A + _L + lane],
                               a1, mask=lane < 9)

            acc0 = jnp.zeros((_L,), jnp.float32)
            acc1 = jnp.zeros((_L,), jnp.float32)
            for a in range(_A):
                w = a0[a] if a < _L else a1[a - _L]
                acc0 = acc0 + w * vrows_v[slot, a // 5, (a % 5) * _L + pi,
                                          pl.ds(0, _L)]
                acc1 = acc1 + w * vrows_v[slot, a // 5, (a % 5) * _L + pi,
                                          pl.ds(_L, _L)]
            out_v[slot, pl.ds(pi * _D, _L)] = acc0
            out_v[slot, pl.ds(pi * _D + _L, _L)] = acc1
            return carry2

        def pair_body2(pj, carry2):
            pair_body(2 * pj, carry2)
            pair_body(2 * pj + 1, carry2)
            return carry2

        lax.fori_loop(0, _L // 2, pair_body2, 0)
        for cp in out_copies(ci, slot):
            cp.start()

    def wait_gathers(ci, slot):
        for cp in gather_copies(ci, slot):
            cp.wait()

    def wait_outputs(ci, slot):
        for cp in out_copies(ci, slot):
            cp.wait()

    n_iters = chunks_per_tile // 2

    mo_copy(0, 0).start()
    mo_copy(1, 1).start()
    fire_chunk(0, 0)

    def loop_body(i, carry):
        cA = 2 * i
        cB = cA + 1
        fire_chunk(cB, 1)
        wait_gathers(cA, 0)

        @pl.when(i >= 1)
        def _():
            wait_outputs(cA, 0)

        compute_chunk(cA, 0)

        @pl.when(i < n_iters - 1)
        def _():
            fire_chunk(cB + 1, 0)

        wait_gathers(cB, 1)

        @pl.when(i >= 1)
        def _():
            wait_outputs(cB, 1)

        compute_chunk(cB, 1)
        return carry

    lax.fori_loop(0, n_iters, loop_body, 0)
    wait_outputs(chunks_per_tile - 2, 0)
    wait_outputs(chunks_per_tile - 1, 1)


@functools.partial(jax.jit, static_argnames=())
def _mf_forward_sc(mo_flat, q_flat, k_flat, v_flat, scale_vec):
    n_pairs = k_flat.shape[0]
    chunks_per_tile = n_pairs // (_NW * _L)
    assert chunks_per_tile % 2 == 0
    mesh = plsc.VectorSubcoreMesh(core_axis_name="c", subcore_axis_name="s")
    body = functools.partial(_sc_body, n_pairs=n_pairs,
                             chunks_per_tile=chunks_per_tile)
    f = pl.kernel(
        body,
        out_type=(
            jax.ShapeDtypeStruct((n_pairs * _D,), jnp.float32),
            jax.ShapeDtypeStruct((n_pairs * _A,), jnp.float32),
        ),
        mesh=mesh,
        compiler_params=pltpu.CompilerParams(
            needs_layout_passes=False,
            use_tc_tiling_on_sc=False,
        ),
        scratch_types=[
            pltpu.VMEM((2, 5, 5 * _L), jnp.int32),
            pltpu.VMEM((2, 5, 5 * _L, _D), jnp.float32),
            pltpu.VMEM((2, 5, 5 * _L, _D), jnp.float32),
            pltpu.VMEM((2, _L, 2), jnp.float32),
            pltpu.VMEM((2, _L * _D), jnp.float32),
            pltpu.VMEM((_L,), jnp.float32),
            pltpu.VMEM((2, _L * _A), jnp.float32),
            pltpu.VMEM((2, _L * _D), jnp.float32),
            pltpu.SemaphoreType.DMA,
            pltpu.SemaphoreType.DMA,
            pltpu.SemaphoreType.DMA,
            pltpu.SemaphoreType.DMA,
            pltpu.SemaphoreType.DMA,
            pltpu.SemaphoreType.DMA,
            pltpu.SemaphoreType.DMA,
            pltpu.SemaphoreType.DMA,
            pltpu.SemaphoreType.DMA,
            pltpu.SemaphoreType.DMA,
            pltpu.SemaphoreType.DMA,
            pltpu.SemaphoreType.DMA,
        ],
    )
    return f(mo_flat, q_flat, k_flat, v_flat, scale_vec)


def kernel(max_offset, q, k, v, H, W, win_r, attn_num, attn_type=1.0,
           scale=1.0):
    B, N, C = q.shape
    h = max_offset.shape[2]
    n_pairs = B * N * h
    mo_flat = max_offset.reshape(n_pairs, 2)
    q_flat = q.reshape(n_pairs * _D)
    k_flat = k.reshape(n_pairs, _D)
    v_flat = v.reshape(n_pairs, _D)
    scale_vec = jnp.full((_L,), 1.0, jnp.float32) * jnp.asarray(
        scale, jnp.float32)
    out_flat, attn_flat = _mf_forward_sc(mo_flat, q_flat, k_flat, v_flat,
                                         scale_vec)
    output = out_flat.reshape(B, N, C)
    attn_out = attn_flat.reshape(B, N, h, _A)
    return output, attn_out
